# Initial kernel scaffold; baseline (speedup 1.0000x reference)
import jax
import jax.numpy as jnp
from jax.experimental import pallas as pl


def kernel(encoder_output, pitch_target, energy_target, pitch_table, energy_table):
    raise NotImplementedError("write your pallas kernel here")


# speedup vs baseline: 3.8442x; 3.8442x over previous
import functools

import jax
import jax.numpy as jnp
from jax import lax
from jax.experimental import pallas as pl
from jax.experimental.pallas import tpu as pltpu
from jax.experimental.pallas import tpu_sc as plsc

NC = 2
NS = 16
L = 16
NW = NC * NS

CHUNK = 128


def _bins_16(raw, v_min, v_max, num_bins):
    div = jnp.float32(num_bins - 1)

    def bval(idx):
        s = idx.astype(jnp.float32) / div
        return jnp.float32(v_min) * (1.0 - s) + jnp.float32(v_max) * s

    v = jnp.clip(raw, v_min, v_max)
    inv_step = (num_bins - 1) / (v_max - v_min)
    e = (v - v_min) * inv_step
    i0 = jnp.minimum((e + 0.5).astype(jnp.int32), num_bins - 1)
    im1 = jnp.maximum(i0 - 1, 0)
    b_lo = bval(im1)
    b_hi = bval(i0)
    one = jnp.ones((L,), jnp.int32)
    zero = jnp.zeros((L,), jnp.int32)
    c = (i0 - 1) + jnp.where(b_lo < v, one, zero) + jnp.where(b_hi < v, one, zero)
    return jnp.clip(c, 0, num_bins - 1)


def _sc_body(enc_hbm, pt_hbm, et_hbm, ptab_hbm, etab_hbm,
             out_hbm, ptv, etv, pbin_v, ebin_v,
             enc_v, prow_v, erow_v, sem0, sem1, *, tokens_per_tile, h):
    wid = lax.axis_index("s") * NC + lax.axis_index("c")
    tile_base = wid * tokens_per_tile

    def chunk_body(g, carry):
        base = tile_base + g * CHUNK
        pltpu.sync_copy(pt_hbm.at[pl.ds(base, CHUNK)], ptv)
        pltpu.sync_copy(et_hbm.at[pl.ds(base, CHUNK)], etv)
        enc_copy = pltpu.async_copy(enc_hbm.at[pl.ds(base, CHUNK)], enc_v, sem1)

        for i in range(CHUNK // L):
            sl = pl.ds(i * L, L)
            pbin_v[sl] = _bins_16(ptv[sl], 50.0, 400.0, 256)
            ebin_v[sl] = _bins_16(etv[sl], 0.0, 1.0, 256)

        gp = pltpu.async_copy(ptab_hbm.at[pbin_v], prow_v, sem0)
        ge = pltpu.async_copy(etab_hbm.at[ebin_v], erow_v, sem0)
        gp.wait()
        ge.wait()
        enc_copy.wait()

        def row_body(r, c2):
            for j in range(h // L):
                sl2 = pl.ds(j * L, L)
                enc_v[r, sl2] = enc_v[r, sl2] + prow_v[r, sl2] + erow_v[r, sl2]
            return c2

        lax.fori_loop(0, CHUNK, row_body, 0, unroll=False)
        pltpu.sync_copy(enc_v, out_hbm.at[pl.ds(base, CHUNK)])
        return carry

    lax.fori_loop(0, tokens_per_tile // CHUNK, chunk_body, 0, unroll=False)


def kernel(encoder_output, pitch_target, energy_target, pitch_table, energy_table):
    b, t, h = encoder_output.shape
    n = b * t
    num_pitch = pitch_table.shape[0]
    num_energy = energy_table.shape[0]
    tokens_per_tile = n // NW

    enc2 = encoder_output.reshape(n, h)
    pt = pitch_target.reshape(n)
    et = energy_target.reshape(n)

    mesh = plsc.VectorSubcoreMesh(
        core_axis_name="c", subcore_axis_name="s",
        num_cores=NC, num_subcores=NS)

    run = pl.kernel(
        functools.partial(_sc_body, tokens_per_tile=tokens_per_tile, h=h),
        out_type=jax.ShapeDtypeStruct((n, h), jnp.float32),
        mesh=mesh,
        scratch_types=[
            pltpu.VMEM((CHUNK,), jnp.float32),
            pltpu.VMEM((CHUNK,), jnp.float32),
            pltpu.VMEM((CHUNK,), jnp.int32),
            pltpu.VMEM((CHUNK,), jnp.int32),
            pltpu.VMEM((CHUNK, h), jnp.float32),
            pltpu.VMEM((CHUNK, h), jnp.float32),
            pltpu.VMEM((CHUNK, h), jnp.float32),
            pltpu.SemaphoreType.DMA,
            pltpu.SemaphoreType.DMA,
        ],
    )
    out2 = run(enc2, pt, et, pitch_table, energy_table)
    expanded_lengths = jnp.full((b,), t, dtype=jnp.int32)
    return (out2.reshape(b, t, h), expanded_lengths)
